# Initial kernel scaffold; baseline (speedup 1.0000x reference)
import jax
import jax.numpy as jnp
from jax.experimental import pallas as pl


def kernel(edges, poke1_embeddings, poke2_embeddings, W1, b1, W2, b2, table_move, table_item, table_ability, table_status, table_edge_type, table_major, table_minor, table_turn, W_boosts, b_boosts, W_damage, b_damage, W_side, b_side):
    raise NotImplementedError("write your pallas kernel here")


# speedup vs baseline: 4.3244x; 4.3244x over previous
import functools

import jax
import jax.numpy as jnp
from jax import lax
from jax.experimental import pallas as pl
from jax.experimental.pallas import tpu as pltpu
from jax.experimental.pallas import tpu_sc as plsc

ENTITY_SIZE = 32
NUM_BINS = 16
NC, NS, LANES = 2, 16, 16
NW = NC * NS

_SIZES = dict(move=1024, item=512, ability=384, status=16, edge_type=16,
              major=32, minor=128, turn=20, boosts=7 * 13, dmg=2047, side=6)
_OFF = {}
_acc = 0
for _k, _v in _SIZES.items():
    _OFF[_k] = _acc
    _acc += _v
ROWS = _acc


def _build_packed_table(table_move, table_item, table_ability, table_status,
                        table_edge_type, table_major, table_minor, table_turn,
                        W_boosts, b_boosts, W_damage, b_damage, W_side, b_side):
    d = ENTITY_SIZE
    vals = jnp.arange(13, dtype=jnp.float32) - 6.0
    boost_rows = (vals[None, :, None] * W_boosts[:, None, :]).reshape(7 * 13, d)
    v = jnp.arange(-1023, 1024, dtype=jnp.int32)
    raw = v / 1023.0
    divisor = 2048.0 / NUM_BINS
    tok = jnp.floor((v + 1023) / divisor)
    tok = jnp.where(v == 0, NUM_BINS + 1, tok)
    onehot = jax.nn.one_hot(tok, NUM_BINS + 1)
    feats = jnp.concatenate([raw[:, None], jnp.abs(raw)[:, None],
                             jnp.sign(v).astype(jnp.float32)[:, None], onehot],
                            axis=-1)
    dmg_rows = feats @ W_damage + (b_damage + b_boosts)[None, :]
    bits = ((jnp.arange(3, dtype=jnp.int32)[:, None]
             & jnp.asarray([1, 2], jnp.int32)[None, :]) != 0).astype(jnp.float32)
    side_rows = jnp.concatenate([jnp.zeros((3, d), jnp.float32),
                                 bits @ W_side + b_side[None, :]], axis=0)
    tab = jnp.concatenate([
        table_move, table_item, table_ability, table_status, table_edge_type,
        table_major, table_minor, table_turn, boost_rows, dmg_rows, side_rows,
    ], axis=0)
    tab_bf = tab.astype(jnp.bfloat16).reshape(ROWS, d // 2, 2)
    return lax.bitcast_convert_type(tab_bf, jnp.int32)


def _make_gather_sum(n_tokens):
    per_w = n_tokens // NW
    chunk = 256
    n_chunks = per_w // chunk
    groups = chunk // LANES
    mesh = plsc.VectorSubcoreMesh(core_axis_name="c", subcore_axis_name="s")

    @functools.partial(
        pl.kernel,
        out_type=jax.ShapeDtypeStruct((n_tokens * 16,), jnp.int32),
        mesh=mesh,
        scratch_types=[
            pltpu.VMEM((ROWS * 16,), jnp.int32),
            pltpu.VMEM((chunk * 19,), jnp.int32),
            pltpu.VMEM((chunk * 16,), jnp.int32),
        ],
        compiler_params=pltpu.CompilerParams(needs_layout_passes=False),
    )
    def gather_sum(table_hbm, edges_hbm, out_hbm, table_v, edges_v, out_v):
        wid = lax.axis_index("s") * NC + lax.axis_index("c")
        pltpu.sync_copy(table_hbm, table_v)
        lanes = lax.iota(jnp.int32, LANES)

        def chunk_body(c, carry):
            base = wid * per_w + c * chunk
            pltpu.sync_copy(edges_hbm.at[pl.ds(base * 19, chunk * 19)],
                            edges_v)

            def group_body(g, carry2):
                tok = lanes + g * LANES
                tok19 = tok * 19

                def fld(f):
                    return plsc.load_gather(edges_v, [tok19 + f])

                has1 = (fld(0) >= 0).astype(jnp.int32)
                rows = [
                    fld(2),
                    _OFF["item"] + fld(3),
                    _OFF["ability"] + fld(4),
                    _OFF["status"] + fld(5),
                    _OFF["major"] + fld(6),
                    _OFF["minor"] + fld(7),
                    _OFF["edge_type"] + fld(8),
                    _OFF["turn"] + fld(17),
                    _OFF["dmg"] + 1023 + fld(16),
                    _OFF["side"] + fld(18) + 3 * has1,
                ]
                for k in range(7):
                    rows.append(_OFF["boosts"] + 13 * k + 6 + fld(9 + k))
                rows = [r * 16 for r in rows]
                tok16 = tok * 16
                for w in range(16):
                    acc = plsc.bitcast(
                        plsc.load_gather(table_v, [rows[0] + w]), jnp.bfloat16)
                    for r in rows[1:]:
                        acc = acc + plsc.bitcast(
                            plsc.load_gather(table_v, [r + w]), jnp.bfloat16)
                    plsc.store_scatter(out_v, [tok16 + w],
                                       plsc.bitcast(acc, jnp.int32))
                return carry2

            lax.fori_loop(0, groups, group_body, 0)
            pltpu.sync_copy(out_v, out_hbm.at[pl.ds(base * 16, chunk * 16)])
            return carry

        lax.fori_loop(0, n_chunks, chunk_body, 0)

    return gather_sum


def _tc_combine(e01, emb1, emb2, gsum_bf, W1, W2, bias):
    n, d = emb1.shape
    tb = 1024
    grid = n // tb

    def body(e01_ref, e1_ref, e2_ref, g_ref, w1_ref, w2_ref, b_ref, out_ref):
        m1 = (e01_ref[:, 0:1] >= 0).astype(jnp.float32)
        m2 = (e01_ref[:, 1:2] >= 0).astype(jnp.float32)
        p1 = jnp.dot(e1_ref[...], w1_ref[...],
                     preferred_element_type=jnp.float32)
        p2 = jnp.dot(e2_ref[...], w2_ref[...],
                     preferred_element_type=jnp.float32)
        out_ref[...] = (m1 * p1 + m2 * p2 + g_ref[...].astype(jnp.float32)
                        + b_ref[...])

    return pl.pallas_call(
        body,
        grid=(grid,),
        in_specs=[
            pl.BlockSpec((tb, 2), lambda i: (i, 0)),
            pl.BlockSpec((tb, d), lambda i: (i, 0)),
            pl.BlockSpec((tb, d), lambda i: (i, 0)),
            pl.BlockSpec((tb, d), lambda i: (i, 0)),
            pl.BlockSpec((d, d), lambda i: (0, 0)),
            pl.BlockSpec((d, d), lambda i: (0, 0)),
            pl.BlockSpec((1, d), lambda i: (0, 0)),
        ],
        out_specs=pl.BlockSpec((tb, d), lambda i: (i, 0)),
        out_shape=jax.ShapeDtypeStruct((n, d), jnp.float32),
    )(e01, emb1, emb2, gsum_bf, W1, W2, bias)


def kernel(edges, poke1_embeddings, poke2_embeddings, W1, b1, W2, b2,
           table_move, table_item, table_ability, table_status,
           table_edge_type, table_major, table_minor, table_turn,
           W_boosts, b_boosts, W_damage, b_damage, W_side, b_side):
    b, t, _ = edges.shape
    d = ENTITY_SIZE
    n = b * t
    packed = _build_packed_table(
        table_move, table_item, table_ability, table_status, table_edge_type,
        table_major, table_minor, table_turn, W_boosts, b_boosts,
        W_damage, b_damage, W_side, b_side)
    edges_flat = edges.reshape(n, 19)
    gsum_i32 = _make_gather_sum(n)(packed.reshape(-1), edges_flat.reshape(-1))
    gsum_bf = lax.bitcast_convert_type(
        gsum_i32.reshape(n, 16), jnp.bfloat16).reshape(n, d)
    out = _tc_combine(
        edges_flat[:, :2],
        poke1_embeddings.reshape(n, d),
        poke2_embeddings.reshape(n, d),
        gsum_bf, W1, W2,
        (b1 + b2)[None, :])
    return out.reshape(b, t, d)


# speedup vs baseline: 5.2007x; 1.2026x over previous
import functools

import jax
import jax.numpy as jnp
from jax import lax
from jax.experimental import pallas as pl
from jax.experimental.pallas import tpu as pltpu
from jax.experimental.pallas import tpu_sc as plsc

ENTITY_SIZE = 32
NUM_BINS = 16
NC, NS, LANES = 2, 16, 16
NW = NC * NS

_SIZES = dict(move=1024, item=512, ability=384, minor=128, se=256, mt=640,
              b01=169, b23=169, b45=169, b6s=78, dmg=2047)
_OFF = {}
_acc = 0
for _k, _v in _SIZES.items():
    _OFF[_k] = _acc
    _acc += _v
ROWS = _acc


def _build_packed_table(table_move, table_item, table_ability, table_status,
                        table_edge_type, table_major, table_minor, table_turn,
                        W_boosts, b_boosts, W_damage, b_damage, W_side, b_side):
    d = ENTITY_SIZE
    vals = jnp.arange(13, dtype=jnp.float32) - 6.0
    boost = vals[None, :, None] * W_boosts[:, None, :]

    def pair(a, b):
        return (a[:, None, :] + b[None, :, :]).reshape(-1, d)

    v = jnp.arange(-1023, 1024, dtype=jnp.int32)
    raw = v / 1023.0
    divisor = 2048.0 / NUM_BINS
    tok = jnp.floor((v + 1023) / divisor)
    tok = jnp.where(v == 0, NUM_BINS + 1, tok)
    onehot = jax.nn.one_hot(tok, NUM_BINS + 1)
    feats = jnp.concatenate([raw[:, None], jnp.abs(raw)[:, None],
                             jnp.sign(v).astype(jnp.float32)[:, None], onehot],
                            axis=-1)
    dmg_rows = feats @ W_damage + (b_damage + b_boosts)[None, :]
    bits = ((jnp.arange(3, dtype=jnp.int32)[:, None]
             & jnp.asarray([1, 2], jnp.int32)[None, :]) != 0).astype(jnp.float32)
    side_rows = jnp.concatenate([jnp.zeros((3, d), jnp.float32),
                                 bits @ W_side + b_side[None, :]], axis=0)
    tab = jnp.concatenate([
        table_move, table_item, table_ability, table_minor,
        pair(table_status, table_edge_type),
        pair(table_major, table_turn),
        pair(boost[0], boost[1]),
        pair(boost[2], boost[3]),
        pair(boost[4], boost[5]),
        pair(boost[6], side_rows),
        dmg_rows,
    ], axis=0)
    tab_bf = tab.astype(jnp.bfloat16).reshape(ROWS, 2, d // 2)
    tab_bf = jnp.swapaxes(tab_bf, 1, 2)
    return lax.bitcast_convert_type(tab_bf, jnp.int32)


def _make_gather_sum(n_tokens):
    per_w = n_tokens // NW
    chunk = 256
    n_chunks = per_w // chunk
    groups = chunk // LANES
    mesh = plsc.VectorSubcoreMesh(core_axis_name="c", subcore_axis_name="s")

    @functools.partial(
        pl.kernel,
        out_type=jax.ShapeDtypeStruct((n_tokens * 16,), jnp.int32),
        mesh=mesh,
        scratch_types=[
            pltpu.VMEM((ROWS * 16,), jnp.int32),
            pltpu.VMEM((chunk * 19,), jnp.int32),
            pltpu.VMEM((chunk * 16,), jnp.int32),
        ],
        compiler_params=pltpu.CompilerParams(needs_layout_passes=False,
                                             disable_bounds_checks=True),
    )
    def gather_sum(table_hbm, edges_hbm, out_hbm, table_v, edges_v, out_v):
        wid = lax.axis_index("s") * NC + lax.axis_index("c")
        pltpu.sync_copy(table_hbm, table_v)
        lanes = lax.iota(jnp.int32, LANES)

        def chunk_body(c, carry):
            base = wid * per_w + c * chunk
            pltpu.sync_copy(edges_hbm.at[pl.ds(base * 19, chunk * 19)],
                            edges_v)

            def group_body(g, carry2):
                tok = lanes + g * LANES
                tok19 = tok * 19

                def fld(f):
                    return plsc.load_gather(edges_v, [tok19 + f])

                has1 = (fld(0) >= 0).astype(jnp.int32)
                rows = [
                    fld(2),
                    _OFF["item"] + fld(3),
                    _OFF["ability"] + fld(4),
                    _OFF["minor"] + fld(7),
                    _OFF["se"] + fld(5) * 16 + fld(8),
                    _OFF["mt"] + fld(6) * 20 + fld(17),
                    _OFF["b01"] + (fld(9) + 6) * 13 + fld(10) + 6,
                    _OFF["b23"] + (fld(11) + 6) * 13 + fld(12) + 6,
                    _OFF["b45"] + (fld(13) + 6) * 13 + fld(14) + 6,
                    _OFF["b6s"] + (fld(15) + 6) * 6 + fld(18) + 3 * has1,
                    _OFF["dmg"] + 1023 + fld(16),
                ]
                rows = [r * 16 for r in rows]
                tok16 = tok * 16
                for w in range(16):
                    terms = [
                        plsc.bitcast(plsc.load_gather(table_v, [r + w]),
                                     jnp.bfloat16)
                        for r in rows
                    ]
                    while len(terms) > 1:
                        terms = ([terms[i] + terms[i + 1]
                                  for i in range(0, len(terms) - 1, 2)]
                                 + ([terms[-1]] if len(terms) % 2 else []))
                    plsc.store_scatter(out_v, [tok16 + w],
                                       plsc.bitcast(terms[0], jnp.int32))
                return carry2

            lax.fori_loop(0, groups, group_body, 0)
            pltpu.sync_copy(out_v, out_hbm.at[pl.ds(base * 16, chunk * 16)])
            return carry

        lax.fori_loop(0, n_chunks, chunk_body, 0)

    return gather_sum


def _tc_combine(e01, emb1, emb2, gsum_bf, W1, W2, bias):
    n, d = emb1.shape
    tb = 1024
    grid = n // tb

    def body(e01_ref, e1_ref, e2_ref, g_ref, w1_ref, w2_ref, b_ref, out_ref):
        m1 = (e01_ref[:, 0:1] >= 0).astype(jnp.float32)
        m2 = (e01_ref[:, 1:2] >= 0).astype(jnp.float32)
        p1 = jnp.dot(e1_ref[...], w1_ref[...],
                     preferred_element_type=jnp.float32)
        p2 = jnp.dot(e2_ref[...], w2_ref[...],
                     preferred_element_type=jnp.float32)
        out_ref[...] = (m1 * p1 + m2 * p2 + g_ref[...].astype(jnp.float32)
                        + b_ref[...])

    return pl.pallas_call(
        body,
        grid=(grid,),
        in_specs=[
            pl.BlockSpec((tb, 2), lambda i: (i, 0)),
            pl.BlockSpec((tb, d), lambda i: (i, 0)),
            pl.BlockSpec((tb, d), lambda i: (i, 0)),
            pl.BlockSpec((tb, d), lambda i: (i, 0)),
            pl.BlockSpec((d, d), lambda i: (0, 0)),
            pl.BlockSpec((d, d), lambda i: (0, 0)),
            pl.BlockSpec((1, d), lambda i: (0, 0)),
        ],
        out_specs=pl.BlockSpec((tb, d), lambda i: (i, 0)),
        out_shape=jax.ShapeDtypeStruct((n, d), jnp.float32),
    )(e01, emb1, emb2, gsum_bf, W1, W2, bias)


def kernel(edges, poke1_embeddings, poke2_embeddings, W1, b1, W2, b2,
           table_move, table_item, table_ability, table_status,
           table_edge_type, table_major, table_minor, table_turn,
           W_boosts, b_boosts, W_damage, b_damage, W_side, b_side):
    b, t, _ = edges.shape
    d = ENTITY_SIZE
    n = b * t
    packed = _build_packed_table(
        table_move, table_item, table_ability, table_status, table_edge_type,
        table_major, table_minor, table_turn, W_boosts, b_boosts,
        W_damage, b_damage, W_side, b_side)
    edges_flat = edges.reshape(n, 19)
    gsum_i32 = _make_gather_sum(n)(packed.reshape(-1), edges_flat.reshape(-1))
    gsum_bf = jnp.swapaxes(
        lax.bitcast_convert_type(gsum_i32.reshape(n, 16), jnp.bfloat16),
        1, 2).reshape(n, d)
    out = _tc_combine(
        edges_flat[:, :2],
        poke1_embeddings.reshape(n, d),
        poke2_embeddings.reshape(n, d),
        gsum_bf, W1, W2,
        (b1 + b2)[None, :])
    return out.reshape(b, t, d)


# speedup vs baseline: 5.6339x; 1.0833x over previous
import functools

import jax
import jax.numpy as jnp
from jax import lax
from jax.experimental import pallas as pl
from jax.experimental.pallas import tpu as pltpu
from jax.experimental.pallas import tpu_sc as plsc

ENTITY_SIZE = 32
NUM_BINS = 16
NC, NS, LANES = 2, 16, 16
NW = NC * NS

_SIZES = dict(move=1024, item=512, ability=384, minor=128, se=256, mt=640,
              b01=169, b23=169, b45=169, b6s=78, dmg=2047)
_OFF = {}
_acc = 0
for _k, _v in _SIZES.items():
    _OFF[_k] = _acc
    _acc += _v
ROWS = _acc


def _build_packed_table(table_move, table_item, table_ability, table_status,
                        table_edge_type, table_major, table_minor, table_turn,
                        W_boosts, b_boosts, W_damage, b_damage, W_side, b_side):
    d = ENTITY_SIZE
    vals = jnp.arange(13, dtype=jnp.float32) - 6.0
    boost = vals[None, :, None] * W_boosts[:, None, :]

    def pair(a, b):
        return (a[:, None, :] + b[None, :, :]).reshape(-1, d)

    v = jnp.arange(-1023, 1024, dtype=jnp.int32)
    raw = v / 1023.0
    divisor = 2048.0 / NUM_BINS
    tok = jnp.floor((v + 1023) / divisor)
    tok = jnp.where(v == 0, NUM_BINS + 1, tok)
    onehot = jax.nn.one_hot(tok, NUM_BINS + 1)
    feats = jnp.concatenate([raw[:, None], jnp.abs(raw)[:, None],
                             jnp.sign(v).astype(jnp.float32)[:, None], onehot],
                            axis=-1)
    dmg_rows = feats @ W_damage + (b_damage + b_boosts)[None, :]
    bits = ((jnp.arange(3, dtype=jnp.int32)[:, None]
             & jnp.asarray([1, 2], jnp.int32)[None, :]) != 0).astype(jnp.float32)
    side_rows = jnp.concatenate([jnp.zeros((3, d), jnp.float32),
                                 bits @ W_side + b_side[None, :]], axis=0)
    tab = jnp.concatenate([
        table_move, table_item, table_ability, table_minor,
        pair(table_status, table_edge_type),
        pair(table_major, table_turn),
        pair(boost[0], boost[1]),
        pair(boost[2], boost[3]),
        pair(boost[4], boost[5]),
        pair(boost[6], side_rows),
        dmg_rows,
    ], axis=0)
    tab_bf = tab.astype(jnp.bfloat16).reshape(ROWS, 2, d // 2)
    tab_bf = jnp.swapaxes(tab_bf, 1, 2)
    return lax.bitcast_convert_type(tab_bf, jnp.int32)


def _make_gather_sum(n_tokens):
    per_w = n_tokens // NW
    chunk = 128
    n_chunks = per_w // chunk
    groups = chunk // LANES
    mesh = plsc.VectorSubcoreMesh(core_axis_name="c", subcore_axis_name="s")

    @functools.partial(
        pl.kernel,
        out_type=jax.ShapeDtypeStruct((n_tokens, 16), jnp.int32),
        mesh=mesh,
        scratch_types=[
            pltpu.VMEM((ROWS * 16,), jnp.int32),
            pltpu.VMEM((chunk, 19), jnp.int32),
            pltpu.VMEM((chunk, 16), jnp.int32),
        ],
        compiler_params=pltpu.CompilerParams(needs_layout_passes=False,
                                             disable_bounds_checks=True),
    )
    def gather_sum(table_hbm, edges_hbm, out_hbm, table_v, edges_v, out_v):
        wid = lax.axis_index("s") * NC + lax.axis_index("c")
        pltpu.sync_copy(table_hbm, table_v)
        lanes = lax.iota(jnp.int32, LANES)

        def chunk_body(c, carry):
            base = wid * per_w + c * chunk
            pltpu.sync_copy(edges_hbm.at[pl.ds(base, chunk)], edges_v)

            def group_body(g, carry2):
                tok = lanes + g * LANES

                def fld(f):
                    return plsc.load_gather(
                        edges_v, [tok, jnp.full((LANES,), f, jnp.int32)])

                has1 = (fld(0) >= 0).astype(jnp.int32)
                rows = [
                    fld(2),
                    _OFF["item"] + fld(3),
                    _OFF["ability"] + fld(4),
                    _OFF["minor"] + fld(7),
                    _OFF["se"] + fld(5) * 16 + fld(8),
                    _OFF["mt"] + fld(6) * 20 + fld(17),
                    _OFF["b01"] + (fld(9) + 6) * 13 + fld(10) + 6,
                    _OFF["b23"] + (fld(11) + 6) * 13 + fld(12) + 6,
                    _OFF["b45"] + (fld(13) + 6) * 13 + fld(14) + 6,
                    _OFF["b6s"] + (fld(15) + 6) * 6 + fld(18) + 3 * has1,
                    _OFF["dmg"] + 1023 + fld(16),
                ]
                rows = [r * 16 for r in rows]
                for w in range(16):
                    wv = jnp.full((LANES,), w, jnp.int32)
                    terms = [
                        plsc.bitcast(plsc.load_gather(table_v, [r + w]),
                                     jnp.bfloat16)
                        for r in rows
                    ]
                    while len(terms) > 1:
                        terms = ([terms[i] + terms[i + 1]
                                  for i in range(0, len(terms) - 1, 2)]
                                 + ([terms[-1]] if len(terms) % 2 else []))
                    plsc.store_scatter(out_v, [tok, wv],
                                       plsc.bitcast(terms[0], jnp.int32))
                return carry2

            lax.fori_loop(0, groups, group_body, 0)
            pltpu.sync_copy(out_v, out_hbm.at[pl.ds(base, chunk)])
            return carry

        lax.fori_loop(0, n_chunks, chunk_body, 0)

    return gather_sum


def _tc_combine(edges2d, emb1, emb2, gsum_i32, W1, W2, bias):
    n, d = emb1.shape
    tb = 1024
    grid = n // tb

    def body(e_ref, e1_ref, e2_ref, g_ref, w1_ref, w2_ref, b_ref, out_ref):
        m1 = (e_ref[:, 0:1] >= 0).astype(jnp.float32)
        m2 = (e_ref[:, 1:2] >= 0).astype(jnp.float32)
        p1 = jnp.dot(e1_ref[...], w1_ref[...],
                     preferred_element_type=jnp.float32)
        p2 = jnp.dot(e2_ref[...], w2_ref[...],
                     preferred_element_type=jnp.float32)
        x = g_ref[...]
        lo = pltpu.bitcast(x << 16, jnp.float32)
        hi = pltpu.bitcast(jnp.bitwise_and(x, jnp.int32(-65536)), jnp.float32)
        g32 = jnp.concatenate([lo, hi], axis=1)
        out_ref[...] = m1 * p1 + m2 * p2 + g32 + b_ref[...]

    return pl.pallas_call(
        body,
        grid=(grid,),
        in_specs=[
            pl.BlockSpec((tb, 19), lambda i: (i, 0)),
            pl.BlockSpec((tb, d), lambda i: (i, 0)),
            pl.BlockSpec((tb, d), lambda i: (i, 0)),
            pl.BlockSpec((tb, 16), lambda i: (i, 0)),
            pl.BlockSpec((d, d), lambda i: (0, 0)),
            pl.BlockSpec((d, d), lambda i: (0, 0)),
            pl.BlockSpec((1, d), lambda i: (0, 0)),
        ],
        out_specs=pl.BlockSpec((tb, d), lambda i: (i, 0)),
        out_shape=jax.ShapeDtypeStruct((n, d), jnp.float32),
    )(edges2d, emb1, emb2, gsum_i32, W1, W2, bias)


def kernel(edges, poke1_embeddings, poke2_embeddings, W1, b1, W2, b2,
           table_move, table_item, table_ability, table_status,
           table_edge_type, table_major, table_minor, table_turn,
           W_boosts, b_boosts, W_damage, b_damage, W_side, b_side):
    b, t, _ = edges.shape
    d = ENTITY_SIZE
    n = b * t
    packed = _build_packed_table(
        table_move, table_item, table_ability, table_status, table_edge_type,
        table_major, table_minor, table_turn, W_boosts, b_boosts,
        W_damage, b_damage, W_side, b_side)
    edges2d = edges.reshape(n, 19)
    gsum_i32 = _make_gather_sum(n)(packed.reshape(-1), edges2d)
    out = _tc_combine(
        edges2d,
        poke1_embeddings.reshape(n, d),
        poke2_embeddings.reshape(n, d),
        gsum_i32, W1, W2,
        (b1 + b2)[None, :])
    return out.reshape(b, t, d)


# speedup vs baseline: 5.9875x; 1.0628x over previous
import functools

import jax
import jax.numpy as jnp
from jax import lax
from jax.experimental import pallas as pl
from jax.experimental.pallas import tpu as pltpu
from jax.experimental.pallas import tpu_sc as plsc

ENTITY_SIZE = 32
NUM_BINS = 16
NC, NS, LANES = 2, 16, 16
NW = NC * NS

_SIZES = dict(move=1024, item=512, ability=384, minor=128, se=256, mt=640,
              b01=169, b23=169, b45=169, b6s=78, dmg=2047)
_OFF = {}
_acc = 0
for _k, _v in _SIZES.items():
    _OFF[_k] = _acc
    _acc += _v
ROWS = _acc


def _build_packed_table(table_move, table_item, table_ability, table_status,
                        table_edge_type, table_major, table_minor, table_turn,
                        W_boosts, b_boosts, W_damage, b_damage, W_side, b_side):
    d = ENTITY_SIZE
    vals = jnp.arange(13, dtype=jnp.float32) - 6.0
    boost = vals[None, :, None] * W_boosts[:, None, :]

    def pair(a, b):
        return (a[:, None, :] + b[None, :, :]).reshape(-1, d)

    v = jnp.arange(-1023, 1024, dtype=jnp.int32)
    raw = v / 1023.0
    divisor = 2048.0 / NUM_BINS
    tok = jnp.floor((v + 1023) / divisor)
    tok = jnp.where(v == 0, NUM_BINS + 1, tok)
    onehot = jax.nn.one_hot(tok, NUM_BINS + 1)
    feats = jnp.concatenate([raw[:, None], jnp.abs(raw)[:, None],
                             jnp.sign(v).astype(jnp.float32)[:, None], onehot],
                            axis=-1)
    dmg_rows = feats @ W_damage + (b_damage + b_boosts)[None, :]
    bits = ((jnp.arange(3, dtype=jnp.int32)[:, None]
             & jnp.asarray([1, 2], jnp.int32)[None, :]) != 0).astype(jnp.float32)
    side_rows = jnp.concatenate([jnp.zeros((3, d), jnp.float32),
                                 bits @ W_side + b_side[None, :]], axis=0)
    tab = jnp.concatenate([
        table_move, table_item, table_ability, table_minor,
        pair(table_status, table_edge_type),
        pair(table_major, table_turn),
        pair(boost[0], boost[1]),
        pair(boost[2], boost[3]),
        pair(boost[4], boost[5]),
        pair(boost[6], side_rows),
        dmg_rows,
    ], axis=0)
    tab_bf = tab.astype(jnp.bfloat16).reshape(ROWS, 2, d // 2)
    tab_bf = jnp.swapaxes(tab_bf, 1, 2)
    return lax.bitcast_convert_type(tab_bf, jnp.int32)


def _make_gather_sum(n_tokens):
    per_w = n_tokens // NW
    chunk = 256
    n_chunks = per_w // chunk
    groups = chunk // LANES
    mesh = plsc.VectorSubcoreMesh(core_axis_name="c", subcore_axis_name="s")

    @functools.partial(
        pl.kernel,
        out_type=jax.ShapeDtypeStruct((n_tokens, 16), jnp.int32),
        mesh=mesh,
        scratch_types=[
            pltpu.VMEM((ROWS * 16,), jnp.int32),
            pltpu.VMEM((chunk * 19,), jnp.int32),
            pltpu.VMEM((chunk, 16), jnp.int32),
        ],
        compiler_params=pltpu.CompilerParams(needs_layout_passes=False,
                                             disable_bounds_checks=True),
    )
    def gather_sum(table_hbm, edges_hbm, out_hbm, table_v, edges_v, out_v):
        wid = lax.axis_index("s") * NC + lax.axis_index("c")
        pltpu.sync_copy(table_hbm, table_v)
        lanes = lax.iota(jnp.int32, LANES)

        def chunk_body(c, carry):
            base = wid * per_w + c * chunk
            pltpu.sync_copy(edges_hbm.at[pl.ds(base * 19, chunk * 19)],
                            edges_v)

            def group_body(g, carry2):
                tok = lanes + g * LANES
                tok19 = tok * 19

                def fld(f):
                    return plsc.load_gather(edges_v, [tok19 + f])

                has1 = (fld(0) >= 0).astype(jnp.int32)
                rows = [
                    fld(2),
                    _OFF["item"] + fld(3),
                    _OFF["ability"] + fld(4),
                    _OFF["minor"] + fld(7),
                    _OFF["se"] + fld(5) * 16 + fld(8),
                    _OFF["mt"] + fld(6) * 20 + fld(17),
                    _OFF["b01"] + (fld(9) + 6) * 13 + fld(10) + 6,
                    _OFF["b23"] + (fld(11) + 6) * 13 + fld(12) + 6,
                    _OFF["b45"] + (fld(13) + 6) * 13 + fld(14) + 6,
                    _OFF["b6s"] + (fld(15) + 6) * 6 + fld(18) + 3 * has1,
                    _OFF["dmg"] + 1023 + fld(16),
                ]
                rows = [r * 16 for r in rows]
                for w in range(16):
                    wv = jnp.full((LANES,), w, jnp.int32)
                    terms = [
                        plsc.bitcast(plsc.load_gather(table_v, [r + w]),
                                     jnp.bfloat16)
                        for r in rows
                    ]
                    while len(terms) > 1:
                        terms = ([terms[i] + terms[i + 1]
                                  for i in range(0, len(terms) - 1, 2)]
                                 + ([terms[-1]] if len(terms) % 2 else []))
                    plsc.store_scatter(out_v, [tok, wv],
                                       plsc.bitcast(terms[0], jnp.int32))
                return carry2

            lax.fori_loop(0, groups, group_body, 0)
            pltpu.sync_copy(out_v, out_hbm.at[pl.ds(base, chunk)])
            return carry

        lax.fori_loop(0, n_chunks, chunk_body, 0)

    return gather_sum


def _tc_combine(edges, emb1, emb2, gsum_i32, W1, W2, bias):
    b, t, d = emb1.shape
    bb = 8
    grid = b // bb
    nb = bb * t

    def body(e_ref, e1_ref, e2_ref, g_ref, w1_ref, w2_ref, b_ref, out_ref):
        m1 = (e_ref[:, :, 0:1] >= 0).astype(jnp.float32)
        m2 = (e_ref[:, :, 1:2] >= 0).astype(jnp.float32)
        p1 = jnp.dot(e1_ref[...].reshape(nb, d), w1_ref[...],
                     preferred_element_type=jnp.float32)
        p2 = jnp.dot(e2_ref[...].reshape(nb, d), w2_ref[...],
                     preferred_element_type=jnp.float32)
        x = g_ref[...]
        lo = pltpu.bitcast(x << 16, jnp.float32)
        hi = pltpu.bitcast(jnp.bitwise_and(x, jnp.int32(-65536)), jnp.float32)
        g32 = jnp.concatenate([lo, hi], axis=1)
        out_ref[...] = (m1 * p1.reshape(bb, t, d) + m2 * p2.reshape(bb, t, d)
                        + (g32 + b_ref[...]).reshape(bb, t, d))

    return pl.pallas_call(
        body,
        grid=(grid,),
        in_specs=[
            pl.BlockSpec((bb, t, 19), lambda i: (i, 0, 0)),
            pl.BlockSpec((bb, t, d), lambda i: (i, 0, 0)),
            pl.BlockSpec((bb, t, d), lambda i: (i, 0, 0)),
            pl.BlockSpec((nb, 16), lambda i: (i, 0)),
            pl.BlockSpec((d, d), lambda i: (0, 0)),
            pl.BlockSpec((d, d), lambda i: (0, 0)),
            pl.BlockSpec((1, d), lambda i: (0, 0)),
        ],
        out_specs=pl.BlockSpec((bb, t, d), lambda i: (i, 0, 0)),
        out_shape=jax.ShapeDtypeStruct((b, t, d), jnp.float32),
    )(edges, emb1, emb2, gsum_i32, W1, W2, bias)


def kernel(edges, poke1_embeddings, poke2_embeddings, W1, b1, W2, b2,
           table_move, table_item, table_ability, table_status,
           table_edge_type, table_major, table_minor, table_turn,
           W_boosts, b_boosts, W_damage, b_damage, W_side, b_side):
    b, t, _ = edges.shape
    d = ENTITY_SIZE
    n = b * t
    packed = _build_packed_table(
        table_move, table_item, table_ability, table_status, table_edge_type,
        table_major, table_minor, table_turn, W_boosts, b_boosts,
        W_damage, b_damage, W_side, b_side)
    gsum_i32 = _make_gather_sum(n)(packed.reshape(-1),
                                   edges.reshape(n * 19))
    return _tc_combine(edges, poke1_embeddings, poke2_embeddings,
                       gsum_i32, W1, W2, (b1 + b2)[None, :])
